# Initial kernel scaffold; baseline (speedup 1.0000x reference)
import jax
import jax.numpy as jnp
from jax.experimental import pallas as pl


def kernel(x, table):
    raise NotImplementedError("write your pallas kernel here")


# speedup vs baseline: 1.4561x; 1.4561x over previous
import jax
import jax.numpy as jnp
from jax.experimental import pallas as pl


def _add_body(x_ref, t_ref, o_ref):
    o_ref[...] = x_ref[...] + t_ref[...][None]


def kernel(x, table):
    b, l, d = x.shape
    BL = 512
    grid = (l // BL, b)
    return pl.pallas_call(
        _add_body,
        grid=grid,
        in_specs=[
            pl.BlockSpec((1, BL, d), lambda i, j: (j, i, 0)),
            pl.BlockSpec((BL, d), lambda i, j: (i, 0)),
        ],
        out_specs=pl.BlockSpec((1, BL, d), lambda i, j: (j, i, 0)),
        out_shape=jax.ShapeDtypeStruct((b, l, d), x.dtype),
    )(x, table)
